# speedup vs baseline: 1.1784x; 1.1784x over previous
import jax
import jax.numpy as jnp
from jax import lax
from jax.experimental import pallas as pl
from jax.experimental.pallas import tpu as pltpu
from jax.experimental.pallas import tpu_sc as plsc

B, N, D = 4, 8192, 2048
H = N // 2
NC, NS = 2, 16
NW = NC * NS
WPP = NW // (B * 2)
ROWS_PER_W = H // WPP
R = 32


def _body(in_hbm, out_hbm, buf, sem):
    wid = lax.axis_index("s") * NC + lax.axis_index("c")
    b = wid // (2 * WPP)
    rem = wid % (2 * WPP)
    j = rem // WPP
    q = rem % WPP

    def step(it, carry):
        r0 = q * ROWS_PER_W + it * R
        pltpu.sync_copy(in_hbm.at[b, pl.ds(j * H + r0, R), :], buf)
        pltpu.sync_copy(buf, out_hbm.at[b, pl.ds(r0, R), j, :])
        return carry

    lax.fori_loop(0, ROWS_PER_W // R, step, 0)


@jax.jit
def kernel(inputs):
    mesh = plsc.VectorSubcoreMesh(
        core_axis_name="c", subcore_axis_name="s", num_cores=NC,
        num_subcores=NS)
    out4 = pl.kernel(
        _body,
        out_type=jax.ShapeDtypeStruct((B, H, 2, D), jnp.float32),
        mesh=mesh,
        scratch_types=[
            pltpu.VMEM((R, D), jnp.float32),
            pltpu.SemaphoreType.DMA,
        ],
    )(inputs)
    return out4.reshape(B, N, D)


# speedup vs baseline: 1.2097x; 1.0266x over previous
import jax
import jax.numpy as jnp
from jax import lax
from jax.experimental import pallas as pl
from jax.experimental.pallas import tpu as pltpu
from jax.experimental.pallas import tpu_sc as plsc

B, N, D = 4, 8192, 2048
H = N // 2
NC, NS = 2, 16
NW = NC * NS
WPP = NW // (B * 2)
ROWS_PER_W = H // WPP
R = 16


def _body(in_hbm, out_hbm, buf0, buf1, r0s, r1s, w0s, w1s):
    wid = lax.axis_index("s") * NC + lax.axis_index("c")
    b = wid // (2 * WPP)
    rem = wid % (2 * WPP)
    j = rem // WPP
    q = rem % WPP
    nsteps = ROWS_PER_W // R
    nhalf = nsteps // 2

    def rd(it, buf, sem):
        r0 = q * ROWS_PER_W + it * R
        return pltpu.async_copy(in_hbm.at[b, pl.ds(j * H + r0, R), :], buf,
                                sem)

    def wr(it, buf, sem):
        r0 = q * ROWS_PER_W + it * R
        return pltpu.async_copy(buf, out_hbm.at[b, pl.ds(r0, R), j, :], sem)

    def wait_rd(buf, sem):
        pltpu.make_async_copy(in_hbm.at[b, pl.ds(0, R), :], buf, sem).wait()

    def wait_wr(buf, sem):
        pltpu.make_async_copy(buf, out_hbm.at[b, pl.ds(0, R), j, :],
                              sem).wait()

    rd(0, buf0, r0s)

    def step(i2, carry):
        it0 = 2 * i2
        wait_rd(buf0, r0s)
        pl.when(i2 > 0)(lambda: wait_wr(buf1, w1s))
        rd(it0 + 1, buf1, r1s)
        wr(it0, buf0, w0s)
        wait_rd(buf1, r1s)

        def refill_buf0():
            wait_wr(buf0, w0s)
            rd(it0 + 2, buf0, r0s)

        pl.when(i2 + 1 < nhalf)(refill_buf0)
        wr(it0 + 1, buf1, w1s)
        return carry

    lax.fori_loop(0, nhalf, step, 0)
    wait_wr(buf0, w0s)
    wait_wr(buf1, w1s)


@jax.jit
def kernel(inputs):
    mesh = plsc.VectorSubcoreMesh(
        core_axis_name="c", subcore_axis_name="s", num_cores=NC,
        num_subcores=NS)
    out4 = pl.kernel(
        _body,
        out_type=jax.ShapeDtypeStruct((B, H, 2, D), jnp.float32),
        mesh=mesh,
        scratch_types=[
            pltpu.VMEM((R, D), jnp.float32),
            pltpu.VMEM((R, D), jnp.float32),
            pltpu.SemaphoreType.DMA,
            pltpu.SemaphoreType.DMA,
            pltpu.SemaphoreType.DMA,
            pltpu.SemaphoreType.DMA,
        ],
    )(inputs)
    return out4.reshape(B, N, D)
